# Initial kernel scaffold; baseline (speedup 1.0000x reference)
import functools

import jax
import jax.numpy as jnp
from jax import lax
from jax.experimental import pallas as pl
from jax.experimental.pallas import tpu as pltpu
from jax.experimental.pallas import tpu_sc as plsc

N_NODES = 10000
D = 128
D_OUT = 64
DEGW = 16
NC, NS = 2, 16
NW = NC * NS
CHUNK = 128
N_ACC = 10240
EPT = 10240
E_PAD = EPT * NW
NB = EPT // CHUNK
RPT = N_ACC // NS


def _sc_agg_body(with_deg, *refs):
    if with_deg:
        (x_hbm, src_hbm, dst_hbm, zf_hbm, zd_hbm, ones_hbm,
         out_hbm, deg_hbm,
         sidx, didx, rows, ones_v, acc_sh, deg_sh, gsem) = refs
    else:
        (x_hbm, src_hbm, dst_hbm, zf_hbm,
         out_hbm,
         sidx, didx, rows, acc_sh, gsem) = refs

    c = lax.axis_index("c")
    s = lax.axis_index("s")
    tile = s * NC + c

    sl = pl.ds(s * RPT, RPT)
    pltpu.sync_copy(zf_hbm.at[sl], acc_sh.at[sl])
    if with_deg:
        pltpu.sync_copy(zd_hbm.at[sl], deg_sh.at[sl])
        pltpu.sync_copy(ones_hbm, ones_v)
    plsc.subcore_barrier()

    def step(b, carry):
        row = tile * NB + b
        pltpu.sync_copy(src_hbm.at[row], sidx)
        pltpu.sync_copy(dst_hbm.at[row], didx)
        pltpu.async_copy(x_hbm.at[sidx], rows, gsem).wait()
        pltpu.sync_copy(rows, acc_sh.at[didx], add=True)
        if with_deg:
            pltpu.sync_copy(ones_v, deg_sh.at[didx], add=True)
        return carry

    lax.fori_loop(0, NB, step, 0)
    plsc.subcore_barrier()

    base = pl.ds(c * N_ACC + s * RPT, RPT)
    pltpu.sync_copy(acc_sh.at[sl], out_hbm.at[base])
    if with_deg:
        pltpu.sync_copy(deg_sh.at[sl], deg_hbm.at[base])


def _make_sc_agg(with_deg):
    mesh = plsc.VectorSubcoreMesh(core_axis_name="c", subcore_axis_name="s")
    out_type = [jax.ShapeDtypeStruct((NC * N_ACC, D), jnp.float32)]
    if with_deg:
        out_type.append(jax.ShapeDtypeStruct((NC * N_ACC, DEGW), jnp.float32))
    scratch = [
        pltpu.VMEM((CHUNK,), jnp.int32),
        pltpu.VMEM((CHUNK,), jnp.int32),
        pltpu.VMEM((CHUNK, D), jnp.float32),
    ]
    if with_deg:
        scratch.append(pltpu.VMEM((CHUNK, DEGW), jnp.float32))
    scratch.append(pltpu.VMEM_SHARED((N_ACC, D), jnp.float32))
    if with_deg:
        scratch.append(pltpu.VMEM_SHARED((N_ACC, DEGW), jnp.float32))
    scratch.append(pltpu.SemaphoreType.DMA)
    return pl.kernel(
        functools.partial(_sc_agg_body, with_deg),
        out_type=out_type,
        mesh=mesh,
        scratch_types=scratch,
    )


_sc_agg_deg = _make_sc_agg(True)
_sc_agg = _make_sc_agg(False)


def _tc_layer1(p0, p1, d0, d1, x, w1l, b1l, w1r, h_ref):
    deg = d0[:, :1] + d1[:, :1]
    invd = 1.0 / jnp.maximum(deg, 1.0)
    agg = p0[...] + p1[...]
    h = (jnp.dot(agg, w1l[...], preferred_element_type=jnp.float32) * invd
         + b1l[...][None, :]
         + jnp.dot(x[...], w1r[...], preferred_element_type=jnp.float32))
    h_ref[...] = jnp.maximum(h, 0.0)


def _tc_layer2(q0, q1, d0, d1, h, w2l, b2l, w2r, wc, bc, h2_ref, logits_ref):
    deg = d0[:, :1] + d1[:, :1]
    invd = 1.0 / jnp.maximum(deg, 1.0)
    agg = q0[...] + q1[...]
    h2 = (jnp.dot(agg, w2l[...], preferred_element_type=jnp.float32) * invd
          + b2l[...][None, :]
          + jnp.dot(h[...], w2r[...], preferred_element_type=jnp.float32))
    h2 = jnp.maximum(h2, 0.0)
    h2_ref[...] = h2
    logits_ref[...] = (jnp.dot(h2, wc[...], preferred_element_type=jnp.float32)
                       + bc[...][None, :])


def kernel(x, edge_index, W1l, b1l, W1r, W2l, b2l, W2r, Wc, bc):
    E = edge_index.shape[1]
    src = edge_index[0]
    dst = edge_index[1]
    pad = E_PAD - E
    pidx = jnp.arange(pad, dtype=jnp.int32)
    src_p = jnp.concatenate([src, pidx % 16]).reshape(-1, CHUNK)
    dst_p = jnp.concatenate(
        [dst, N_NODES + pidx % (N_ACC - N_NODES)]).reshape(-1, CHUNK)

    zf = jnp.zeros((N_ACC, D), jnp.float32)
    zd = jnp.zeros((N_ACC, DEGW), jnp.float32)
    ones = jnp.ones((CHUNK, DEGW), jnp.float32)

    aggx, degf = _sc_agg_deg(x, src_p, dst_p, zf, zd, ones)
    p0 = aggx[:N_NODES]
    p1 = aggx[N_ACC:N_ACC + N_NODES]
    d0 = degf[:N_NODES]
    d1 = degf[N_ACC:N_ACC + N_NODES]

    h = pl.pallas_call(
        _tc_layer1,
        out_shape=jax.ShapeDtypeStruct((N_NODES, D), jnp.float32),
    )(p0, p1, d0, d1, x, W1l, b1l, W1r)

    (aggh,) = _sc_agg(h, src_p, dst_p, zf)
    q0 = aggh[:N_NODES]
    q1 = aggh[N_ACC:N_ACC + N_NODES]

    h2, logits = pl.pallas_call(
        _tc_layer2,
        out_shape=(jax.ShapeDtypeStruct((N_NODES, D), jnp.float32),
                   jax.ShapeDtypeStruct((N_NODES, D_OUT), jnp.float32)),
    )(q0, q1, d0, d1, h, W2l, b2l, W2r, Wc, bc)

    return (logits, h2)


# speedup vs baseline: 5.5712x; 5.5712x over previous
import jax
import jax.numpy as jnp
from jax import lax
from jax.experimental import pallas as pl
from jax.experimental.pallas import tpu as pltpu
from jax.experimental.pallas import tpu_sc as plsc

N_NODES = 10000
D = 128
D_OUT = 64
NC, NS = 2, 16
NW = NC * NS
CHUNK = 128
N_ACC = 10240
EPT = 10240
E_PAD = EPT * NW
NB = EPT // CHUNK
RPT = N_ACC // NS


def _sc_agg_body(x_hbm, src_hbm, dst_hbm, zf_hbm, out_hbm,
                 sidx, didx, rows, acc_sh, gsem):
    c = lax.axis_index("c")
    s = lax.axis_index("s")
    tile = s * NC + c

    pltpu.sync_copy(zf_hbm, rows)
    for k in range(RPT // CHUNK):
        pltpu.sync_copy(rows, acc_sh.at[pl.ds(s * RPT + k * CHUNK, CHUNK)])
    plsc.subcore_barrier()

    def step(b, carry):
        base = tile * EPT + b * CHUNK
        pltpu.sync_copy(src_hbm.at[pl.ds(base, CHUNK)], sidx)
        pltpu.sync_copy(dst_hbm.at[pl.ds(base, CHUNK)], didx)
        pltpu.async_copy(x_hbm.at[sidx], rows, gsem).wait()
        pltpu.sync_copy(rows, acc_sh.at[didx], add=True)
        return carry

    lax.fori_loop(0, NB, step, 0)
    plsc.subcore_barrier()

    for k in range(RPT // CHUNK):
        sl = pl.ds(s * RPT + k * CHUNK, CHUNK)
        osl = pl.ds(c * N_ACC + s * RPT + k * CHUNK, CHUNK)
        pltpu.sync_copy(acc_sh.at[sl], rows)
        pltpu.sync_copy(rows, out_hbm.at[osl])


def _sc_deg_body(dst_hbm, zf_hbm, ones_hbm, out_hbm,
                 didx, rows, acc_sh):
    c = lax.axis_index("c")
    s = lax.axis_index("s")
    tile = s * NC + c

    pltpu.sync_copy(zf_hbm, rows)
    for k in range(RPT // CHUNK):
        pltpu.sync_copy(rows, acc_sh.at[pl.ds(s * RPT + k * CHUNK, CHUNK)])
    pltpu.sync_copy(ones_hbm, rows)
    plsc.subcore_barrier()

    def step(b, carry):
        base = tile * EPT + b * CHUNK
        pltpu.sync_copy(dst_hbm.at[pl.ds(base, CHUNK)], didx)
        pltpu.sync_copy(rows, acc_sh.at[didx], add=True)
        return carry

    lax.fori_loop(0, NB, step, 0)
    plsc.subcore_barrier()

    for k in range(RPT // CHUNK):
        sl = pl.ds(s * RPT + k * CHUNK, CHUNK)
        osl = pl.ds(c * N_ACC + s * RPT + k * CHUNK, CHUNK)
        pltpu.sync_copy(acc_sh.at[sl], rows)
        pltpu.sync_copy(rows, out_hbm.at[osl])


_SC_MESH = plsc.VectorSubcoreMesh(core_axis_name="c", subcore_axis_name="s")

_sc_agg = pl.kernel(
    _sc_agg_body,
    out_type=[jax.ShapeDtypeStruct((NC * N_ACC, D), jnp.float32)],
    mesh=_SC_MESH,
    scratch_types=[
        pltpu.VMEM((CHUNK,), jnp.int32),
        pltpu.VMEM((CHUNK,), jnp.int32),
        pltpu.VMEM((CHUNK, D), jnp.float32),
        pltpu.VMEM_SHARED((N_ACC, D), jnp.float32),
        pltpu.SemaphoreType.DMA,
    ],
)

_sc_deg = pl.kernel(
    _sc_deg_body,
    out_type=[jax.ShapeDtypeStruct((NC * N_ACC, D), jnp.float32)],
    mesh=_SC_MESH,
    scratch_types=[
        pltpu.VMEM((CHUNK,), jnp.int32),
        pltpu.VMEM((CHUNK, D), jnp.float32),
        pltpu.VMEM_SHARED((N_ACC, D), jnp.float32),
    ],
)


def _tc_layer1(p0, p1, d0, d1, x, w1l, b1l, w1r, h_ref):
    deg = d0[:, :1] + d1[:, :1]
    invd = 1.0 / jnp.maximum(deg, 1.0)
    agg = p0[...] + p1[...]
    h = (jnp.dot(agg, w1l[...], preferred_element_type=jnp.float32) * invd
         + b1l[...][None, :]
         + jnp.dot(x[...], w1r[...], preferred_element_type=jnp.float32))
    h_ref[...] = jnp.maximum(h, 0.0)


def _tc_layer2(q0, q1, d0, d1, h, w2l, b2l, w2r, wc, bc, h2_ref, logits_ref):
    deg = d0[:, :1] + d1[:, :1]
    invd = 1.0 / jnp.maximum(deg, 1.0)
    agg = q0[...] + q1[...]
    h2 = (jnp.dot(agg, w2l[...], preferred_element_type=jnp.float32) * invd
          + b2l[...][None, :]
          + jnp.dot(h[...], w2r[...], preferred_element_type=jnp.float32))
    h2 = jnp.maximum(h2, 0.0)
    h2_ref[...] = h2
    logits_ref[...] = (jnp.dot(h2, wc[...], preferred_element_type=jnp.float32)
                       + bc[...][None, :])


def kernel(x, edge_index, W1l, b1l, W1r, W2l, b2l, W2r, Wc, bc):
    E = edge_index.shape[1]
    src = edge_index[0]
    dst = edge_index[1]
    pad = E_PAD - E
    pidx = jnp.arange(pad, dtype=jnp.int32)
    src_p = jnp.concatenate([src, pidx % 16])
    dst_p = jnp.concatenate([dst, N_NODES + pidx % (N_ACC - N_NODES)])

    zf = jnp.zeros((CHUNK, D), jnp.float32)
    ones = jnp.ones((CHUNK, D), jnp.float32)

    (degf,) = _sc_deg(dst_p, zf, ones)
    d0 = degf[:N_NODES]
    d1 = degf[N_ACC:N_ACC + N_NODES]

    (aggx,) = _sc_agg(x, src_p, dst_p, zf)
    p0 = aggx[:N_NODES]
    p1 = aggx[N_ACC:N_ACC + N_NODES]

    h = pl.pallas_call(
        _tc_layer1,
        out_shape=jax.ShapeDtypeStruct((N_NODES, D), jnp.float32),
    )(p0, p1, d0, d1, x, W1l, b1l, W1r)

    (aggh,) = _sc_agg(h, src_p, dst_p, zf)
    q0 = aggh[:N_NODES]
    q1 = aggh[N_ACC:N_ACC + N_NODES]

    h2, logits = pl.pallas_call(
        _tc_layer2,
        out_shape=(jax.ShapeDtypeStruct((N_NODES, D), jnp.float32),
                   jax.ShapeDtypeStruct((N_NODES, D_OUT), jnp.float32)),
    )(q0, q1, d0, d1, h, W2l, b2l, W2r, Wc, bc)

    return (logits, h2)


# speedup vs baseline: 10.1273x; 1.8178x over previous
import jax
import jax.numpy as jnp
from jax import lax
from jax.experimental import pallas as pl
from jax.experimental.pallas import tpu as pltpu
from jax.experimental.pallas import tpu_sc as plsc

N_NODES = 10000
D = 128
D_OUT = 64
NC, NS = 2, 16
NW = NC * NS
CHUNK = 128
N_ACC = 10240
EPT = 10240
E_PAD = EPT * NW
NB = EPT // CHUNK
RPT = N_ACC // NS


def _flush_acc(acc_sh, out_hbm, c, s, rows0, rows1, fsem0, fsem1):
    bufs = (rows0, rows1)
    sems = (fsem0, fsem1)
    nk = RPT // CHUNK
    for k in range(nk):
        slot = k % 2
        sl = pl.ds(s * RPT + k * CHUNK, CHUNK)
        osl = pl.ds(c * N_ACC + s * RPT + k * CHUNK, CHUNK)
        if k >= 2:
            psl = pl.ds(c * N_ACC + s * RPT + (k - 2) * CHUNK, CHUNK)
            pltpu.make_async_copy(bufs[slot], out_hbm.at[psl], sems[slot]).wait()
        pltpu.sync_copy(acc_sh.at[sl], bufs[slot])
        pltpu.async_copy(bufs[slot], out_hbm.at[osl], sems[slot])
    for k in range(max(nk - 2, 0), nk):
        slot = k % 2
        osl = pl.ds(c * N_ACC + s * RPT + k * CHUNK, CHUNK)
        pltpu.make_async_copy(bufs[slot], out_hbm.at[osl], sems[slot]).wait()


NB2 = NB // 2


def _sc_agg_body(x_hbm, src_hbm, dst_hbm, zf_hbm, out_hbm,
                 sidx_h, didx_h, rows0, rows1, acc_sh,
                 gsem0, gsem1, ssem0, ssem1):
    c = lax.axis_index("c")
    s = lax.axis_index("s")
    tile = s * NC + c

    pltpu.sync_copy(zf_hbm, rows0)
    for k in range(RPT // CHUNK):
        pltpu.sync_copy(rows0, acc_sh.at[pl.ds(s * RPT + k * CHUNK, CHUNK)])
    plsc.subcore_barrier()

    rows = (rows0, rows1)
    gsems = (gsem0, gsem1)
    ssems = (ssem0, ssem1)

    for half in range(2):
        pltpu.sync_copy(
            src_hbm.at[pl.ds(tile * NB + half * NB2, NB2)], sidx_h)
        pltpu.sync_copy(
            dst_hbm.at[pl.ds(tile * NB + half * NB2, NB2)], didx_h)

        pltpu.async_copy(x_hbm.at[sidx_h.at[0]], rows0, gsem0)
        pltpu.async_copy(x_hbm.at[sidx_h.at[1]], rows1, gsem1)

        def step(i, carry):
            for slot in (0, 1):
                b = 2 * i + slot
                pltpu.make_async_copy(
                    x_hbm.at[sidx_h.at[b]], rows[slot], gsems[slot]).wait()
                pltpu.async_copy(
                    rows[slot], acc_sh.at[didx_h.at[b]], ssems[slot],
                    add=True)
                pltpu.make_async_copy(
                    rows[slot], acc_sh.at[didx_h.at[b]], ssems[slot]).wait()
                pltpu.async_copy(
                    x_hbm.at[sidx_h.at[b + 2]], rows[slot], gsems[slot])
            return carry

        lax.fori_loop(0, NB2 // 2 - 1, step, 0)
        for slot in (0, 1):
            b = NB2 - 2 + slot
            pltpu.make_async_copy(
                x_hbm.at[sidx_h.at[b]], rows[slot], gsems[slot]).wait()
            pltpu.async_copy(
                rows[slot], acc_sh.at[didx_h.at[b]], ssems[slot], add=True)
            pltpu.make_async_copy(
                rows[slot], acc_sh.at[didx_h.at[b]], ssems[slot]).wait()

    plsc.subcore_barrier()

    _flush_acc(acc_sh, out_hbm, c, s, rows0, rows1, gsem0, gsem1)


_DEG_DEPTH = 8


def _sc_deg_body(dst_hbm, zf_hbm, ones_hbm, out_hbm,
                 didx_h, rows0, rows1, acc_sh, ssem, fsem0, fsem1):
    c = lax.axis_index("c")
    s = lax.axis_index("s")
    tile = s * NC + c

    pltpu.sync_copy(zf_hbm, rows0)
    for k in range(RPT // CHUNK):
        pltpu.sync_copy(rows0, acc_sh.at[pl.ds(s * RPT + k * CHUNK, CHUNK)])
    pltpu.sync_copy(ones_hbm, rows0)
    plsc.subcore_barrier()

    for half in range(2):
        pltpu.sync_copy(
            dst_hbm.at[pl.ds(tile * NB + half * NB2, NB2)], didx_h)

        def step(b, carry):
            pltpu.async_copy(rows0, acc_sh.at[didx_h.at[b]], ssem, add=True)

            @pl.when(b >= _DEG_DEPTH)
            def _():
                pltpu.make_async_copy(
                    rows0, acc_sh.at[didx_h.at[b - _DEG_DEPTH]], ssem).wait()
            return carry

        lax.fori_loop(0, NB2, step, 0)
        for k in range(_DEG_DEPTH):
            pltpu.make_async_copy(
                rows0, acc_sh.at[didx_h.at[NB2 - _DEG_DEPTH + k]],
                ssem).wait()

    plsc.subcore_barrier()

    _flush_acc(acc_sh, out_hbm, c, s, rows0, rows1, fsem0, fsem1)


_SC_MESH = plsc.VectorSubcoreMesh(core_axis_name="c", subcore_axis_name="s")

_sc_agg = pl.kernel(
    _sc_agg_body,
    out_type=[jax.ShapeDtypeStruct((NC * N_ACC, D), jnp.float32)],
    mesh=_SC_MESH,
    scratch_types=[
        pltpu.VMEM((NB2, CHUNK), jnp.int32),
        pltpu.VMEM((NB2, CHUNK), jnp.int32),
        pltpu.VMEM((CHUNK, D), jnp.float32),
        pltpu.VMEM((CHUNK, D), jnp.float32),
        pltpu.VMEM_SHARED((N_ACC, D), jnp.float32),
        pltpu.SemaphoreType.DMA,
        pltpu.SemaphoreType.DMA,
        pltpu.SemaphoreType.DMA,
        pltpu.SemaphoreType.DMA,
    ],
)

_sc_deg = pl.kernel(
    _sc_deg_body,
    out_type=[jax.ShapeDtypeStruct((NC * N_ACC, D), jnp.float32)],
    mesh=_SC_MESH,
    scratch_types=[
        pltpu.VMEM((NB2, CHUNK), jnp.int32),
        pltpu.VMEM((CHUNK, D), jnp.float32),
        pltpu.VMEM((CHUNK, D), jnp.float32),
        pltpu.VMEM_SHARED((N_ACC, D), jnp.float32),
        pltpu.SemaphoreType.DMA,
        pltpu.SemaphoreType.DMA,
        pltpu.SemaphoreType.DMA,
    ],
)


def _tc_layer1(p0, p1, d0, d1, x, w1l, b1l, w1r, h_ref):
    deg = d0[:, :1] + d1[:, :1]
    invd = 1.0 / jnp.maximum(deg, 1.0)
    agg = p0[...] + p1[...]
    h = (jnp.dot(agg, w1l[...], preferred_element_type=jnp.float32) * invd
         + b1l[...][None, :]
         + jnp.dot(x[...], w1r[...], preferred_element_type=jnp.float32))
    h_ref[...] = jnp.maximum(h, 0.0)


def _tc_layer2(q0, q1, d0, d1, h, w2l, b2l, w2r, wc, bc, h2_ref, logits_ref):
    deg = d0[:, :1] + d1[:, :1]
    invd = 1.0 / jnp.maximum(deg, 1.0)
    agg = q0[...] + q1[...]
    h2 = (jnp.dot(agg, w2l[...], preferred_element_type=jnp.float32) * invd
          + b2l[...][None, :]
          + jnp.dot(h[...], w2r[...], preferred_element_type=jnp.float32))
    h2 = jnp.maximum(h2, 0.0)
    h2_ref[...] = h2
    logits_ref[...] = (jnp.dot(h2, wc[...], preferred_element_type=jnp.float32)
                       + bc[...][None, :])


def kernel(x, edge_index, W1l, b1l, W1r, W2l, b2l, W2r, Wc, bc):
    E = edge_index.shape[1]
    src = edge_index[0]
    dst = edge_index[1]
    pad = E_PAD - E
    pidx = jnp.arange(pad, dtype=jnp.int32)
    src_p = jnp.concatenate([src, pidx % 16]).reshape(NW * NB, CHUNK)
    dst_p = jnp.concatenate(
        [dst, N_NODES + pidx % (N_ACC - N_NODES)]).reshape(NW * NB, CHUNK)

    zf = jnp.zeros((CHUNK, D), jnp.float32)
    ones = jnp.ones((CHUNK, D), jnp.float32)

    (degf,) = _sc_deg(dst_p, zf, ones)
    d0 = degf[:N_NODES]
    d1 = degf[N_ACC:N_ACC + N_NODES]

    (aggx,) = _sc_agg(x, src_p, dst_p, zf)
    p0 = aggx[:N_NODES]
    p1 = aggx[N_ACC:N_ACC + N_NODES]

    h = pl.pallas_call(
        _tc_layer1,
        out_shape=jax.ShapeDtypeStruct((N_NODES, D), jnp.float32),
    )(p0, p1, d0, d1, x, W1l, b1l, W1r)

    (aggh,) = _sc_agg(h, src_p, dst_p, zf)
    q0 = aggh[:N_NODES]
    q1 = aggh[N_ACC:N_ACC + N_NODES]

    h2, logits = pl.pallas_call(
        _tc_layer2,
        out_shape=(jax.ShapeDtypeStruct((N_NODES, D), jnp.float32),
                   jax.ShapeDtypeStruct((N_NODES, D_OUT), jnp.float32)),
    )(q0, q1, d0, d1, h, W2l, b2l, W2r, Wc, bc)

    return (logits, h2)


# speedup vs baseline: 10.5454x; 1.0413x over previous
import jax
import jax.numpy as jnp
from jax import lax
from jax.experimental import pallas as pl
from jax.experimental.pallas import tpu as pltpu
from jax.experimental.pallas import tpu_sc as plsc

N_NODES = 10000
D = 128
D_OUT = 64
NC, NS = 2, 16
NW = NC * NS
CHUNK = 128
N_ACC = 10240
EPT = 10240
E_PAD = EPT * NW
NB = EPT // CHUNK
RPT = N_ACC // NS


def _flush_acc(acc_sh, out_hbm, c, s, rows0, rows1, fsem0, fsem1):
    bufs = (rows0, rows1)
    sems = (fsem0, fsem1)
    nk = RPT // CHUNK
    for k in range(nk):
        slot = k % 2
        sl = pl.ds(s * RPT + k * CHUNK, CHUNK)
        osl = pl.ds(c * N_ACC + s * RPT + k * CHUNK, CHUNK)
        if k >= 2:
            psl = pl.ds(c * N_ACC + s * RPT + (k - 2) * CHUNK, CHUNK)
            pltpu.make_async_copy(bufs[slot], out_hbm.at[psl], sems[slot]).wait()
        pltpu.sync_copy(acc_sh.at[sl], bufs[slot])
        pltpu.async_copy(bufs[slot], out_hbm.at[osl], sems[slot])
    for k in range(max(nk - 2, 0), nk):
        slot = k % 2
        osl = pl.ds(c * N_ACC + s * RPT + k * CHUNK, CHUNK)
        pltpu.make_async_copy(bufs[slot], out_hbm.at[osl], sems[slot]).wait()


NB2 = NB // 2


def _sc_agg_body(x_hbm, src_hbm, dst_hbm, zf_hbm, out_hbm,
                 sidx_h, didx_h, rows0, rows1, acc_sh,
                 gsem0, gsem1, ssem0, ssem1):
    c = lax.axis_index("c")
    s = lax.axis_index("s")
    tile = s * NC + c

    pltpu.sync_copy(zf_hbm, rows0)
    for k in range(RPT // CHUNK):
        pltpu.sync_copy(rows0, acc_sh.at[pl.ds(s * RPT + k * CHUNK, CHUNK)])
    plsc.subcore_barrier()

    rows = (rows0, rows1)
    gsems = (gsem0, gsem1)
    ssems = (ssem0, ssem1)

    for half in range(2):
        pltpu.sync_copy(
            src_hbm.at[pl.ds(tile * NB + half * NB2, NB2)], sidx_h)
        pltpu.sync_copy(
            dst_hbm.at[pl.ds(tile * NB + half * NB2, NB2)], didx_h)

        pltpu.async_copy(x_hbm.at[sidx_h.at[0]], rows0, gsem0)
        pltpu.async_copy(x_hbm.at[sidx_h.at[1]], rows1, gsem1)

        def step(i, carry):
            for slot in (0, 1):
                b = 2 * i + slot
                pltpu.make_async_copy(
                    x_hbm.at[sidx_h.at[b]], rows[slot], gsems[slot]).wait()
                pltpu.async_copy(
                    rows[slot], acc_sh.at[didx_h.at[b]], ssems[slot],
                    add=True)
                pltpu.make_async_copy(
                    rows[slot], acc_sh.at[didx_h.at[b]], ssems[slot]).wait()
                pltpu.async_copy(
                    x_hbm.at[sidx_h.at[b + 2]], rows[slot], gsems[slot])
            return carry

        lax.fori_loop(0, NB2 // 2 - 1, step, 0)
        for slot in (0, 1):
            b = NB2 - 2 + slot
            pltpu.make_async_copy(
                x_hbm.at[sidx_h.at[b]], rows[slot], gsems[slot]).wait()
            pltpu.async_copy(
                rows[slot], acc_sh.at[didx_h.at[b]], ssems[slot], add=True)
            pltpu.make_async_copy(
                rows[slot], acc_sh.at[didx_h.at[b]], ssems[slot]).wait()

    plsc.subcore_barrier()

    _flush_acc(acc_sh, out_hbm, c, s, rows0, rows1, gsem0, gsem1)


_DEG_DEPTH = 8


def _sc_deg_body(dst_hbm, zf_hbm, ones_hbm, out_hbm,
                 didx_h, rows0, rows1, acc_sh, ssem, fsem0, fsem1):
    c = lax.axis_index("c")
    s = lax.axis_index("s")
    tile = s * NC + c

    pltpu.sync_copy(zf_hbm, rows0)
    for k in range(RPT // CHUNK):
        pltpu.sync_copy(rows0, acc_sh.at[pl.ds(s * RPT + k * CHUNK, CHUNK)])
    pltpu.sync_copy(ones_hbm, rows0)
    plsc.subcore_barrier()

    for half in range(2):
        pltpu.sync_copy(
            dst_hbm.at[pl.ds(tile * NB + half * NB2, NB2)], didx_h)

        def step(b, carry):
            pltpu.async_copy(rows0, acc_sh.at[didx_h.at[b]], ssem, add=True)

            @pl.when(b >= _DEG_DEPTH)
            def _():
                pltpu.make_async_copy(
                    rows0, acc_sh.at[didx_h.at[b - _DEG_DEPTH]], ssem).wait()
            return carry

        lax.fori_loop(0, NB2, step, 0)
        for k in range(_DEG_DEPTH):
            pltpu.make_async_copy(
                rows0, acc_sh.at[didx_h.at[NB2 - _DEG_DEPTH + k]],
                ssem).wait()

    plsc.subcore_barrier()

    _flush_acc(acc_sh, out_hbm, c, s, rows0, rows1, fsem0, fsem1)


_SC_MESH = plsc.VectorSubcoreMesh(core_axis_name="c", subcore_axis_name="s")

_sc_agg = pl.kernel(
    _sc_agg_body,
    out_type=[jax.ShapeDtypeStruct((NC * N_ACC, D), jnp.float32)],
    mesh=_SC_MESH,
    scratch_types=[
        pltpu.VMEM((NB2, CHUNK), jnp.int32),
        pltpu.VMEM((NB2, CHUNK), jnp.int32),
        pltpu.VMEM((CHUNK, D), jnp.float32),
        pltpu.VMEM((CHUNK, D), jnp.float32),
        pltpu.VMEM_SHARED((N_ACC, D), jnp.float32),
        pltpu.SemaphoreType.DMA,
        pltpu.SemaphoreType.DMA,
        pltpu.SemaphoreType.DMA,
        pltpu.SemaphoreType.DMA,
    ],
)

_sc_deg = pl.kernel(
    _sc_deg_body,
    out_type=[jax.ShapeDtypeStruct((NC * N_ACC, D), jnp.float32)],
    mesh=_SC_MESH,
    scratch_types=[
        pltpu.VMEM((NB2, CHUNK), jnp.int32),
        pltpu.VMEM((CHUNK, D), jnp.float32),
        pltpu.VMEM((CHUNK, D), jnp.float32),
        pltpu.VMEM_SHARED((N_ACC, D), jnp.float32),
        pltpu.SemaphoreType.DMA,
        pltpu.SemaphoreType.DMA,
        pltpu.SemaphoreType.DMA,
    ],
)


def _tc_layer1(aggx, degf, x, w1l, b1l, w1r, h_ref, invd_ref):
    deg = degf[0:N_NODES, 0:1] + degf[N_ACC:N_ACC + N_NODES, 0:1]
    invd = 1.0 / jnp.maximum(deg, 1.0)
    invd_ref[...] = invd
    agg = aggx[0:N_NODES, :] + aggx[N_ACC:N_ACC + N_NODES, :]
    h = (jnp.dot(agg, w1l[...], preferred_element_type=jnp.float32) * invd
         + b1l[...][None, :]
         + jnp.dot(x[...], w1r[...], preferred_element_type=jnp.float32))
    h_ref[...] = jnp.maximum(h, 0.0)


def _tc_layer2(aggh, invd_ref, h, w2l, b2l, w2r, wc, bc, h2_ref, logits_ref):
    invd = invd_ref[...]
    agg = aggh[0:N_NODES, :] + aggh[N_ACC:N_ACC + N_NODES, :]
    h2 = (jnp.dot(agg, w2l[...], preferred_element_type=jnp.float32) * invd
          + b2l[...][None, :]
          + jnp.dot(h[...], w2r[...], preferred_element_type=jnp.float32))
    h2 = jnp.maximum(h2, 0.0)
    h2_ref[...] = h2
    logits_ref[...] = (jnp.dot(h2, wc[...], preferred_element_type=jnp.float32)
                       + bc[...][None, :])


def kernel(x, edge_index, W1l, b1l, W1r, W2l, b2l, W2r, Wc, bc):
    E = edge_index.shape[1]
    src = edge_index[0]
    dst = edge_index[1]
    pad = E_PAD - E
    pidx = jnp.arange(pad, dtype=jnp.int32)
    src_p = jnp.concatenate([src, pidx % 16]).reshape(NW * NB, CHUNK)
    dst_p = jnp.concatenate(
        [dst, N_NODES + pidx % (N_ACC - N_NODES)]).reshape(NW * NB, CHUNK)

    zf = jnp.zeros((CHUNK, D), jnp.float32)
    ones = jnp.ones((CHUNK, D), jnp.float32)

    (degf,) = _sc_deg(dst_p, zf, ones)
    (aggx,) = _sc_agg(x, src_p, dst_p, zf)

    h, invd = pl.pallas_call(
        _tc_layer1,
        out_shape=(jax.ShapeDtypeStruct((N_NODES, D), jnp.float32),
                   jax.ShapeDtypeStruct((N_NODES, 1), jnp.float32)),
    )(aggx, degf, x, W1l, b1l, W1r)

    (aggh,) = _sc_agg(h, src_p, dst_p, zf)

    h2, logits = pl.pallas_call(
        _tc_layer2,
        out_shape=(jax.ShapeDtypeStruct((N_NODES, D), jnp.float32),
                   jax.ShapeDtypeStruct((N_NODES, D_OUT), jnp.float32)),
    )(aggh, invd, h, W2l, b2l, W2r, Wc, bc)

    return (logits, h2)
